# baseline (speedup 1.0000x reference)
import jax
import jax.numpy as jnp
from jax import lax
from jax.experimental import pallas as pl
from jax.experimental.pallas import tpu as pltpu

_B, _C, _H, _W = 32, 512, 64, 64
_HW = _H * _W
_MID = max(16, _C // 16)
_NE = 64
_K = 2


def _router_body(x_ref, w1_ref, b1_ref, w2_ref, b2_ref,
                 wout_ref, iout_ref, acc_ref):
    i = pl.program_id(0)
    blk = x_ref[0]
    acc_ref[i, :] = jnp.sum(blk, axis=1) * (1.0 / _HW)

    @pl.when(i == _B - 1)
    def _finish():
        g = acc_ref[...]
        h = jnp.dot(g, w1_ref[...],
                    preferred_element_type=jnp.float32) + b1_ref[...]
        h = h * jax.nn.sigmoid(h)
        logits = jnp.dot(h, w2_ref[...],
                         preferred_element_type=jnp.float32) + b2_ref[...]

        idx = lax.broadcasted_iota(jnp.int32, (_B, _NE), 1)
        m1 = jnp.max(logits, axis=1, keepdims=True)
        i1 = jnp.min(jnp.where(logits == m1, idx, _NE), axis=1, keepdims=True)
        masked = jnp.where(idx == i1, -jnp.inf, logits)
        m2 = jnp.max(masked, axis=1, keepdims=True)
        i2 = jnp.min(jnp.where(masked == m2, idx, _NE), axis=1, keepdims=True)

        e = jnp.exp(m2 - m1)
        denom = 1.0 + e
        wout_ref[...] = jnp.concatenate([1.0 / denom, e / denom], axis=1)
        iout_ref[...] = jnp.concatenate([i1, i2], axis=1)


def kernel(x, W1, b1, W2, b2):
    x3 = x.reshape(_B, _C, _HW)
    b1r = b1.reshape(1, _MID)
    b2r = b2.reshape(1, _NE)

    wout, iout = pl.pallas_call(
        _router_body,
        grid=(_B,),
        in_specs=[
            pl.BlockSpec((1, _C, _HW), lambda i: (i, 0, 0)),
            pl.BlockSpec((_C, _MID), lambda i: (0, 0)),
            pl.BlockSpec((1, _MID), lambda i: (0, 0)),
            pl.BlockSpec((_MID, _NE), lambda i: (0, 0)),
            pl.BlockSpec((1, _NE), lambda i: (0, 0)),
        ],
        out_specs=[
            pl.BlockSpec((_B, _K), lambda i: (0, 0)),
            pl.BlockSpec((_B, _K), lambda i: (0, 0)),
        ],
        out_shape=[
            jax.ShapeDtypeStruct((_B, _K), jnp.float32),
            jax.ShapeDtypeStruct((_B, _K), jnp.int32),
        ],
        scratch_shapes=[pltpu.VMEM((_B, _C), jnp.float32)],
        compiler_params=pltpu.CompilerParams(
            dimension_semantics=("arbitrary",),
        ),
    )(x3, W1, b1r, W2, b2r)
    return wout, iout
